# baseline (speedup 1.0000x reference)
import jax
import jax.numpy as jnp
from jax import lax
from jax.experimental import pallas as pl
from jax.experimental.pallas import tpu as pltpu
from jax.experimental.pallas import tpu_sc as plsc

VOCAB = 100000
D = 128
B = 16384
NCTX = 4
NNEG = 5
NIDX = 1 + NCTX + NNEG

NC = 2
NS = 16
NW = NC * NS
BPW = B // NW
K = 64
NCHUNK = BPW // K
NSET = 3


def _sc_body(cen_hbm, ctx_hbm, neg_hbm_idx, in_emb, out_emb, pos_hbm, neg_hbm,
             cen_b, ctxr_b, negr_b, cidx_b, nidx_b, u_b, vs_b, ns_b,
             pos_sb, neg_sb, raw_sems, g_sems, out_sems):
    cid = lax.axis_index("c")
    sid = lax.axis_index("s")
    wid = sid * NC + cid
    base0 = wid * BPW
    iota16 = jnp.arange(16, dtype=jnp.int32)
    zf = jnp.zeros((16,), jnp.float32)

    raw_cps = {}
    g_cps = {}
    out_cps = {}

    def fire_raw(t):
        b = base0 + t * K
        raw_cps[t] = [
            pltpu.async_copy(cen_hbm.at[pl.ds(b, K)], cen_b[t % NSET],
                             raw_sems[0]),
            pltpu.async_copy(ctx_hbm.at[pl.ds(b, K)], ctxr_b, raw_sems[0]),
            pltpu.async_copy(neg_hbm_idx.at[pl.ds(b, K)], negr_b,
                             raw_sems[0]),
        ]

    def extract(t):
        s = t % NSET
        for g in range(K // 16):
            row = iota16 + (g * 16)
            for c in range(NCTX):
                col = jnp.full((16,), c, jnp.int32)
                cidx_b[s][c][pl.ds(g * 16, 16)] = plsc.load_gather(
                    ctxr_b, [row, col])
            for c in range(NNEG):
                col = jnp.full((16,), c, jnp.int32)
                nidx_b[s][c][pl.ds(g * 16, 16)] = plsc.load_gather(
                    negr_b, [row, col])

    def zero_set(s):
        def z(k, carry):
            for j in range(D // 16):
                vs_b[s][k, pl.ds(16 * j, 16)] = zf
                ns_b[s][k, pl.ds(16 * j, 16)] = zf
            return carry
        lax.fori_loop(0, K, z, 0)

    def fire_gathers(t):
        s = t % NSET
        for cp in raw_cps.pop(t):
            cp.wait()
        extract(t)
        cps = [pltpu.async_copy(in_emb.at[cen_b[s]], u_b[s], g_sems[s])]
        for c in range(NCTX):
            cps.append(pltpu.async_copy(out_emb.at[cidx_b[s][c]], vs_b[s],
                                        g_sems[s], add=True))
        for c in range(NNEG):
            cps.append(pltpu.async_copy(out_emb.at[nidx_b[s][c]], ns_b[s],
                                        g_sems[s], add=True))
        g_cps[t] = cps

    def compute(t):
        s = t % NSET
        o = t % 2
        for cp in g_cps.pop(t):
            cp.wait()
        if t - 2 in out_cps:
            for cp in out_cps.pop(t - 2):
                cp.wait()
        u_v, vs_v, ns_v = u_b[s], vs_b[s], ns_b[s]

        def grp(g, carry):
            sp_v = jnp.zeros((16,), jnp.float32)
            sn_v = jnp.zeros((16,), jnp.float32)
            for i in range(16):
                k = g * 16 + i
                accp = u_v[k, pl.ds(0, 16)] * vs_v[k, pl.ds(0, 16)]
                accn = u_v[k, pl.ds(0, 16)] * ns_v[k, pl.ds(0, 16)]
                vs_v[k, pl.ds(0, 16)] = zf
                ns_v[k, pl.ds(0, 16)] = zf
                for j in range(1, D // 16):
                    uu = u_v[k, pl.ds(16 * j, 16)]
                    accp = accp + uu * vs_v[k, pl.ds(16 * j, 16)]
                    accn = accn + uu * ns_v[k, pl.ds(16 * j, 16)]
                    vs_v[k, pl.ds(16 * j, 16)] = zf
                    ns_v[k, pl.ds(16 * j, 16)] = zf
                sp = jnp.sum(accp) * (1.0 / NCTX)
                sn = jnp.sum(accn) * (1.0 / NNEG)
                sp_v = jnp.where(iota16 == i, sp, sp_v)
                sn_v = jnp.where(iota16 == i, sn, sn_v)
            pos_sb[o][pl.ds(g * 16, 16)] = sp_v
            neg_sb[o][pl.ds(g * 16, 16)] = sn_v
            return carry

        lax.fori_loop(0, K // 16, grp, 0)
        b = base0 + t * K
        out_cps[t] = [
            pltpu.async_copy(pos_sb[o], pos_hbm.at[pl.ds(b, K)], out_sems[o]),
            pltpu.async_copy(neg_sb[o], neg_hbm.at[pl.ds(b, K)], out_sems[o]),
        ]

    fire_raw(0)
    zero_set(0)
    fire_gathers(0)
    fire_raw(1)
    zero_set(1)
    fire_gathers(1)
    fire_raw(2)
    zero_set(2)
    for t in range(NCHUNK):
        if t + 2 < NCHUNK:
            fire_gathers(t + 2)
        if t + 3 < NCHUNK:
            fire_raw(t + 3)
        compute(t)
    for t in sorted(out_cps):
        for cp in out_cps[t]:
            cp.wait()


def _scores_sc(centers, context, neg_context, in_emb, out_emb):
    mesh = plsc.VectorSubcoreMesh(core_axis_name="c", subcore_axis_name="s",
                                  num_cores=NC, num_subcores=NS)
    f32 = jnp.float32
    i32 = jnp.int32
    run = pl.kernel(
        _sc_body,
        out_type=(jax.ShapeDtypeStruct((B,), f32),
                  jax.ShapeDtypeStruct((B,), f32)),
        mesh=mesh,
        compiler_params=pltpu.CompilerParams(needs_layout_passes=False),
        scratch_types=[
            [pltpu.VMEM((K,), i32) for _ in range(NSET)],
            pltpu.VMEM((K, NCTX), i32),
            pltpu.VMEM((K, NNEG), i32),
            [[pltpu.VMEM((K,), i32) for _ in range(NCTX)]
             for _ in range(NSET)],
            [[pltpu.VMEM((K,), i32) for _ in range(NNEG)]
             for _ in range(NSET)],
            [pltpu.VMEM((K, D), f32) for _ in range(NSET)],
            [pltpu.VMEM((K, D), f32) for _ in range(NSET)],
            [pltpu.VMEM((K, D), f32) for _ in range(NSET)],
            [pltpu.VMEM((K,), f32) for _ in range(2)],
            [pltpu.VMEM((K,), f32) for _ in range(2)],
            [pltpu.SemaphoreType.DMA],
            [pltpu.SemaphoreType.DMA for _ in range(NSET)],
            [pltpu.SemaphoreType.DMA for _ in range(2)],
        ],
    )
    return run(centers, context, neg_context, in_emb, out_emb)


def _loss_body(pos_ref, neg_ref, o_ref):
    pos = pos_ref[...]
    neg = neg_ref[...]
    loss = jax.nn.log_sigmoid(pos) + jax.nn.log_sigmoid(-neg)
    o_ref[0, 0] = -jnp.mean(loss)


def _loss_tc(pos, neg):
    out = pl.pallas_call(
        _loss_body,
        out_shape=jax.ShapeDtypeStruct((1, 1), jnp.float32),
        in_specs=[pl.BlockSpec(memory_space=pltpu.VMEM),
                  pl.BlockSpec(memory_space=pltpu.VMEM)],
        out_specs=pl.BlockSpec(memory_space=pltpu.SMEM),
    )(pos.reshape(128, 128), neg.reshape(128, 128))
    return out[0, 0]


@jax.jit
def kernel(centers, context, neg_context, in_emb, out_emb):
    centers = centers.astype(jnp.int32)
    context = context.astype(jnp.int32)
    neg_context = neg_context.astype(jnp.int32)
    pos, neg = _scores_sc(centers, context, neg_context, in_emb, out_emb)
    return _loss_tc(pos, neg)


# speedup vs baseline: 1.1369x; 1.1369x over previous
import jax
import jax.numpy as jnp
from jax import lax
from jax.experimental import pallas as pl
from jax.experimental.pallas import tpu as pltpu
from jax.experimental.pallas import tpu_sc as plsc

VOCAB = 100000
D = 128
B = 16384
NCTX = 4
NNEG = 5
NIDX = 1 + NCTX + NNEG

NC = 2
NS = 16
NW = NC * NS
BPW = B // NW
K = 128
NCHUNK = BPW // K
NSET = 2


def _sc_body(idx_hbm, in_emb, out_emb, pos_hbm, neg_hbm,
             idx_b, u_b, vs_b, ns_b, pos_sb, neg_sb,
             raw_sems, g_sems, out_sems):
    cid = lax.axis_index("c")
    sid = lax.axis_index("s")
    wid = sid * NC + cid
    base0 = wid * BPW
    iota16 = jnp.arange(16, dtype=jnp.int32)
    zf = jnp.zeros((16,), jnp.float32)

    raw_cps = {}
    g_cps = {}
    out_cps = {}

    def fire_raw(t):
        s = t % NSET
        b = base0 + t * K
        raw_cps[t] = [
            pltpu.async_copy(idx_hbm.at[r, pl.ds(b, K)], idx_b[s][r],
                             raw_sems[s])
            for r in range(NIDX)
        ]

    def zero_buf(buf):
        def z(k, carry):
            for j in range(D // 16):
                buf[k, pl.ds(16 * j, 16)] = zf
            return carry
        lax.fori_loop(0, K, z, 0)

    def fire_gathers(t):
        s = t % NSET
        for cp in raw_cps.pop(t):
            cp.wait()
        cps = [pltpu.async_copy(in_emb.at[idx_b[s][0]], u_b[s], g_sems[s])]
        for r in range(1, 1 + NCTX):
            cps.append(pltpu.async_copy(out_emb.at[idx_b[s][r]], vs_b[s],
                                        g_sems[s], add=True))
        for r in range(1 + NCTX, NIDX):
            cps.append(pltpu.async_copy(out_emb.at[idx_b[s][r]], ns_b[s],
                                        g_sems[s], add=True))
        g_cps[t] = cps

    def compute(t):
        s = t % NSET
        o = t % 2
        for cp in g_cps.pop(t):
            cp.wait()
        if t - 2 in out_cps:
            for cp in out_cps.pop(t - 2):
                cp.wait()
        u_v, vs_v, ns_v = u_b[s], vs_b[s], ns_b[s]

        def grp(g, carry):
            sp_v = jnp.zeros((16,), jnp.float32)
            sn_v = jnp.zeros((16,), jnp.float32)
            for i in range(16):
                k = g * 16 + i
                accp = u_v[k, pl.ds(0, 16)] * vs_v[k, pl.ds(0, 16)]
                accn = u_v[k, pl.ds(0, 16)] * ns_v[k, pl.ds(0, 16)]
                vs_v[k, pl.ds(0, 16)] = zf
                ns_v[k, pl.ds(0, 16)] = zf
                for j in range(1, D // 16):
                    uu = u_v[k, pl.ds(16 * j, 16)]
                    accp = accp + uu * vs_v[k, pl.ds(16 * j, 16)]
                    accn = accn + uu * ns_v[k, pl.ds(16 * j, 16)]
                    vs_v[k, pl.ds(16 * j, 16)] = zf
                    ns_v[k, pl.ds(16 * j, 16)] = zf
                sp = jnp.sum(accp) * (1.0 / NCTX)
                sn = jnp.sum(accn) * (1.0 / NNEG)
                sp_v = jnp.where(iota16 == i, sp, sp_v)
                sn_v = jnp.where(iota16 == i, sn, sn_v)
            pos_sb[o][pl.ds(g * 16, 16)] = sp_v
            neg_sb[o][pl.ds(g * 16, 16)] = sn_v
            return carry

        lax.fori_loop(0, K // 16, grp, 0)
        b = base0 + t * K
        out_cps[t] = [
            pltpu.async_copy(pos_sb[o], pos_hbm.at[pl.ds(b, K)], out_sems[o]),
            pltpu.async_copy(neg_sb[o], neg_hbm.at[pl.ds(b, K)], out_sems[o]),
        ]

    fire_raw(0)
    fire_raw(1)
    for cp in raw_cps.pop(0):
        cp.wait()
    cps0 = [pltpu.async_copy(in_emb.at[idx_b[0][0]], u_b[0], g_sems[0])]
    zero_buf(vs_b[0])
    for r in range(1, 1 + NCTX):
        cps0.append(pltpu.async_copy(out_emb.at[idx_b[0][r]], vs_b[0],
                                     g_sems[0], add=True))
    zero_buf(ns_b[0])
    for r in range(1 + NCTX, NIDX):
        cps0.append(pltpu.async_copy(out_emb.at[idx_b[0][r]], ns_b[0],
                                     g_sems[0], add=True))
    g_cps[0] = cps0
    zero_buf(vs_b[1])
    zero_buf(ns_b[1])
    for t in range(NCHUNK):
        if t + 1 < NCHUNK:
            fire_gathers(t + 1)
        compute(t)
        if t + 2 < NCHUNK:
            fire_raw(t + 2)
    for t in sorted(out_cps):
        for cp in out_cps[t]:
            cp.wait()


def _scores_sc(idx_all, in_emb, out_emb):
    mesh = plsc.VectorSubcoreMesh(core_axis_name="c", subcore_axis_name="s",
                                  num_cores=NC, num_subcores=NS)
    f32 = jnp.float32
    i32 = jnp.int32
    run = pl.kernel(
        _sc_body,
        out_type=(jax.ShapeDtypeStruct((B,), f32),
                  jax.ShapeDtypeStruct((B,), f32)),
        mesh=mesh,
        compiler_params=pltpu.CompilerParams(needs_layout_passes=False),
        scratch_types=[
            [[pltpu.VMEM((K,), i32) for _ in range(NIDX)]
             for _ in range(NSET)],
            [pltpu.VMEM((K, D), f32) for _ in range(NSET)],
            [pltpu.VMEM((K, D), f32) for _ in range(NSET)],
            [pltpu.VMEM((K, D), f32) for _ in range(NSET)],
            [pltpu.VMEM((K,), f32) for _ in range(2)],
            [pltpu.VMEM((K,), f32) for _ in range(2)],
            [pltpu.SemaphoreType.DMA for _ in range(NSET)],
            [pltpu.SemaphoreType.DMA for _ in range(NSET)],
            [pltpu.SemaphoreType.DMA for _ in range(2)],
        ],
    )
    return run(idx_all, in_emb, out_emb)


def _loss_body(pos_ref, neg_ref, o_ref):
    pos = pos_ref[...]
    neg = neg_ref[...]
    loss = jax.nn.log_sigmoid(pos) + jax.nn.log_sigmoid(-neg)
    o_ref[0, 0] = -jnp.mean(loss)


def _loss_tc(pos, neg):
    out = pl.pallas_call(
        _loss_body,
        out_shape=jax.ShapeDtypeStruct((1, 1), jnp.float32),
        in_specs=[pl.BlockSpec(memory_space=pltpu.VMEM),
                  pl.BlockSpec(memory_space=pltpu.VMEM)],
        out_specs=pl.BlockSpec(memory_space=pltpu.SMEM),
    )(pos.reshape(128, 128), neg.reshape(128, 128))
    return out[0, 0]


@jax.jit
def kernel(centers, context, neg_context, in_emb, out_emb):
    centers = centers.astype(jnp.int32)
    context = context.astype(jnp.int32)
    neg_context = neg_context.astype(jnp.int32)
    idx_all = jnp.concatenate(
        [centers[None, :], context.T, neg_context.T], axis=0)
    pos, neg = _scores_sc(idx_all, in_emb, out_emb)
    return _loss_tc(pos, neg)
